# baseline (speedup 1.0000x reference)
import functools

import jax
import jax.numpy as jnp
from jax.experimental import pallas as pl


def _select_kernel(x_ref, p_ref, b_ref, g_ref, v_ref):
    x = x_ref[0]
    p_full = p_ref[...]
    bias = b_ref[0, 0]

    y = jax.lax.dot_general(
        x.astype(jnp.bfloat16), p_full.astype(jnp.bfloat16),
        (((1,), (0,)), ((), ())),
        preferred_element_type=jnp.float32)
    wcol = y[:, 0:1]
    s_col = jax.nn.sigmoid(wcol + bias)
    s_row = s_col.T

    gt = (s_col > s_row).astype(jnp.int32)
    jlt = (jax.lax.broadcasted_iota(jnp.int32, (512, 512), 0)
           < jax.lax.broadcasted_iota(jnp.int32, (512, 512), 1))
    eq = ((s_col == s_row) & jlt).astype(jnp.int32)
    rank_row = jnp.sum(gt + eq, axis=0, keepdims=True)

    r_iota = jax.lax.broadcasted_iota(jnp.int32, (128, 512), 0)
    m = (rank_row == r_iota).astype(jnp.float32)

    g_ref[0] = jax.lax.dot_general(
        m, x, (((1,), (0,)), ((), ())),
        preferred_element_type=jnp.float32,
        precision=jax.lax.Precision.HIGHEST)
    v_ref[0] = jnp.sum(m * s_row, axis=1, keepdims=True)


def _scale_kernel(g_ref, v_ref, o_ref):
    o_ref[0] = g_ref[0] * v_ref[...]


@jax.jit
def kernel(node_fts, rel_edges, W, b):
    del rel_edges
    B, N, D = node_fts.shape
    K = 128
    b2 = b.reshape(1, 1).astype(jnp.float32)
    p = jnp.pad(W.reshape(D, 1), ((0, 0), (0, D - 1)))

    gathered, vals = pl.pallas_call(
        _select_kernel,
        grid=(B,),
        in_specs=[
            pl.BlockSpec((1, N, D), lambda i: (i, 0, 0)),
            pl.BlockSpec((D, D), lambda i: (0, 0)),
            pl.BlockSpec((1, 1), lambda i: (0, 0)),
        ],
        out_specs=[
            pl.BlockSpec((1, K, D), lambda i: (i, 0, 0)),
            pl.BlockSpec((1, K, 1), lambda i: (i, 0, 0)),
        ],
        out_shape=[
            jax.ShapeDtypeStruct((B, K, D), jnp.float32),
            jax.ShapeDtypeStruct((B, K, 1), jnp.float32),
        ],
    )(node_fts, p, b2)

    value = vals.reshape(B, K)

    out = pl.pallas_call(
        _scale_kernel,
        grid=(B,),
        in_specs=[
            pl.BlockSpec((1, K, D), lambda i: (i, 0, 0)),
            pl.BlockSpec((K, D), lambda i: (0, 0)),
        ],
        out_specs=pl.BlockSpec((1, K, D), lambda i: (i, 0, 0)),
        out_shape=jax.ShapeDtypeStruct((B, K, D), jnp.float32),
    )(gathered, value)
    return out
